# Initial kernel scaffold; baseline (speedup 1.0000x reference)
import functools
import jax
import jax.numpy as jnp
from jax import lax
from jax.experimental import pallas as pl
from jax.experimental.pallas import tpu as pltpu
from jax.experimental.pallas import tpu_sc as plsc

NC = 2
NS = 16
NW = NC * NS
K = 80

_mesh = plsc.VectorSubcoreMesh(core_axis_name="c", subcore_axis_name="s")


def _make_deg_kernel(nch, npad):
    spt = npad // NS

    @functools.partial(
        pl.kernel,
        out_type=jax.ShapeDtypeStruct((NC, npad), jnp.float32),
        mesh=_mesh,
        scratch_types=[
            pltpu.VMEM((nch, K), jnp.int32),
            pltpu.VMEM((K,), jnp.float32),
            pltpu.VMEM((spt,), jnp.float32),
            pltpu.VMEM_SHARED((npad,), jnp.float32),
        ],
    )
    def deg_kernel(dst_hbm, out_hbm, idx_v, ones_v, z_v, acc):
        c = lax.axis_index("c")
        s = lax.axis_index("s")
        w = s * NC + c
        pltpu.sync_copy(dst_hbm.at[w], idx_v)

        for i in range(K // 16):
            ones_v[pl.ds(i * 16, 16)] = jnp.ones((16,), jnp.float32)

        def zrow(i, carry):
            z_v[pl.ds(i * 16, 16)] = jnp.zeros((16,), jnp.float32)
            return carry

        lax.fori_loop(0, spt // 16, zrow, 0)
        pltpu.sync_copy(z_v, acc.at[pl.ds(s * spt, spt)])
        plsc.subcore_barrier()

        def step(j, carry):
            pltpu.sync_copy(ones_v, acc.at[idx_v.at[j]], add=True)
            return carry

        lax.fori_loop(0, nch, step, 0)
        plsc.subcore_barrier()
        pltpu.sync_copy(acc.at[pl.ds(s * spt, spt)],
                        out_hbm.at[c, pl.ds(s * spt, spt)])

    return deg_kernel


def _make_agg_kernel(n, d, nch):
    rpt = n // NS
    zr = min(rpt, 125)
    assert rpt % zr == 0

    @functools.partial(
        pl.kernel,
        out_type=jax.ShapeDtypeStruct((NC, n, d), jnp.float32),
        mesh=_mesh,
        scratch_types=[
            pltpu.VMEM((2, nch, K), jnp.int32),
            pltpu.VMEM((2, K, d), jnp.float32),
            pltpu.VMEM((zr, d), jnp.float32),
            pltpu.VMEM_SHARED((n, d), jnp.float32),
            pltpu.SemaphoreType.DMA,
            pltpu.SemaphoreType.DMA,
        ],
    )
    def agg_kernel(hp_hbm, src_hbm, dst_hbm, out_hbm,
                   idx_v, rows_v, z_v, acc, sem0, sem1):
        c = lax.axis_index("c")
        s = lax.axis_index("s")
        w = s * NC + c
        pltpu.sync_copy(src_hbm.at[w], idx_v.at[0])
        pltpu.sync_copy(dst_hbm.at[w], idx_v.at[1])

        def zrow(i, carry):
            for jv in range(d // 16):
                z_v[i, pl.ds(jv * 16, 16)] = jnp.zeros((16,), jnp.float32)
            return carry

        lax.fori_loop(0, zr, zrow, 0)
        for t in range(rpt // zr):
            pltpu.sync_copy(z_v, acc.at[pl.ds(s * rpt + t * zr, zr)])
        plsc.subcore_barrier()

        sems = (sem0, sem1)
        pltpu.async_copy(hp_hbm.at[idx_v.at[0, 0]], rows_v.at[0], sem0)

        def pair(i, carry):
            for b in range(2):
                jj = 2 * i + b
                pltpu.async_copy(hp_hbm.at[idx_v.at[0, jj + 1]],
                                 rows_v.at[1 - b], sems[1 - b])
                pltpu.make_async_copy(hp_hbm.at[idx_v.at[0, jj]],
                                      rows_v.at[b], sems[b]).wait()
                pltpu.sync_copy(rows_v.at[b], acc.at[idx_v.at[1, jj]],
                                add=True)
            return carry

        lax.fori_loop(0, (nch - 1) // 2, pair, 0)
        last = nch - 1
        lb = last % 2
        pltpu.make_async_copy(hp_hbm.at[idx_v.at[0, last]],
                              rows_v.at[lb], sems[lb]).wait()
        pltpu.sync_copy(rows_v.at[lb], acc.at[idx_v.at[1, last]], add=True)

        plsc.subcore_barrier()
        pltpu.sync_copy(acc.at[pl.ds(s * rpt, rpt)],
                        out_hbm.at[c, pl.ds(s * rpt, rpt)])

    return agg_kernel


def _mm_scale_body(x_ref, w_ref, dinv_ref, o_ref):
    o_ref[...] = jnp.dot(x_ref[...], w_ref[...],
                         preferred_element_type=jnp.float32) * dinv_ref[...]


def _layer_body(s0_ref, s1_ref, hp_ref, dinv_ref, b_ref, w_ref, o_ref):
    z = (s0_ref[...] + s1_ref[...] + hp_ref[...]) * dinv_ref[...] + b_ref[...]
    z = jnp.maximum(z, 0.0)
    o_ref[...] = jnp.dot(z, w_ref[...],
                         preferred_element_type=jnp.float32) * dinv_ref[...]


def _final_body(s0_ref, s1_ref, hp_ref, dinv_ref, b_ref, o_ref):
    o_ref[...] = ((s0_ref[...] + s1_ref[...] + hp_ref[...]) * dinv_ref[...]
                  + b_ref[...])


def _row_spec(r, d):
    return pl.BlockSpec((r, d), lambda i: (i, 0))


def _full_spec(shape):
    return pl.BlockSpec(shape, lambda i: tuple(0 for _ in shape))


def kernel(x, edge_index, W1, b1, W2, b2, W3, b3):
    n, d_in = x.shape
    hid = W1.shape[1]
    out_d = W3.shape[1]
    e = edge_index.shape[1]
    nch = e // (NW * K)
    assert nch * NW * K == e and n % NS == 0

    src3 = edge_index[0].reshape(NW, nch, K)
    dst3 = edge_index[1].reshape(NW, nch, K)

    npad = ((n + 16 * NS - 1) // (16 * NS)) * (16 * NS)
    degp = _make_deg_kernel(nch, npad)(dst3)
    deg = 1.0 + degp[0, :n] + degp[1, :n]
    dinv = lax.rsqrt(deg).reshape(n, 1)

    agg = _make_agg_kernel(n, hid, nch)

    r = 1000 if n % 1000 == 0 else n
    grid = (n // r,)

    mm_scale = pl.pallas_call(
        _mm_scale_body,
        grid=grid,
        in_specs=[_row_spec(r, d_in), _full_spec((d_in, hid)),
                  _row_spec(r, 1)],
        out_specs=_row_spec(r, hid),
        out_shape=jax.ShapeDtypeStruct((n, hid), jnp.float32),
    )

    def layer(w_next):
        return pl.pallas_call(
            _layer_body,
            grid=grid,
            in_specs=[_row_spec(r, hid), _row_spec(r, hid),
                      _row_spec(r, hid), _row_spec(r, 1),
                      _full_spec((1, hid)), _full_spec((hid, w_next))],
            out_specs=_row_spec(r, w_next),
            out_shape=jax.ShapeDtypeStruct((n, w_next), jnp.float32),
        )

    final = pl.pallas_call(
        _final_body,
        grid=grid,
        in_specs=[_row_spec(r, out_d), _row_spec(r, out_d),
                  _row_spec(r, out_d), _row_spec(r, 1),
                  _full_spec((1, out_d))],
        out_specs=_row_spec(r, out_d),
        out_shape=jax.ShapeDtypeStruct((n, out_d), jnp.float32),
    )

    h1p = mm_scale(x, W1, dinv)
    s1 = agg(h1p, src3, dst3)
    h2p = layer(hid)(s1[0], s1[1], h1p, dinv, b1.reshape(1, hid), W2)
    s2 = agg(h2p, src3, dst3)
    h3p = layer(out_d)(s2[0], s2[1], h2p, dinv, b2.reshape(1, hid), W3)
    s3 = agg(h3p, src3, dst3)
    return final(s3[0], s3[1], h3p, dinv, b3.reshape(1, out_d))


# speedup vs baseline: 20.5699x; 20.5699x over previous
import functools
import jax
import jax.numpy as jnp
from jax import lax
from jax.experimental import pallas as pl
from jax.experimental.pallas import tpu as pltpu
from jax.experimental.pallas import tpu_sc as plsc

NC = 2
NS = 16
NW = NC * NS
K = 80

_mesh = plsc.VectorSubcoreMesh(core_axis_name="c", subcore_axis_name="s")


def _make_deg_kernel(nch, npad):
    spt = npad // NS

    @functools.partial(
        pl.kernel,
        out_type=jax.ShapeDtypeStruct((NC * npad,), jnp.float32),
        mesh=_mesh,
        scratch_types=[
            pltpu.VMEM((nch, K), jnp.int32),
            pltpu.VMEM((K,), jnp.float32),
            pltpu.VMEM((spt,), jnp.float32),
            pltpu.VMEM_SHARED((npad,), jnp.float32),
        ],
    )
    def deg_kernel(dst_hbm, out_hbm, idx_v, ones_v, z_v, acc):
        c = lax.axis_index("c")
        s = lax.axis_index("s")
        w = s * NC + c
        pltpu.sync_copy(dst_hbm.at[w], idx_v)

        for i in range(K // 16):
            ones_v[pl.ds(i * 16, 16)] = jnp.ones((16,), jnp.float32)

        def zrow(i, carry):
            z_v[pl.ds(i * 16, 16)] = jnp.zeros((16,), jnp.float32)
            return carry

        lax.fori_loop(0, spt // 16, zrow, 0)
        pltpu.sync_copy(z_v, acc.at[pl.ds(s * spt, spt)])
        plsc.subcore_barrier()

        def step(j, carry):
            pltpu.sync_copy(ones_v, acc.at[idx_v.at[j]], add=True)
            return carry

        lax.fori_loop(0, nch, step, 0)
        plsc.subcore_barrier()
        pltpu.sync_copy(acc.at[pl.ds(s * spt, spt)],
                        out_hbm.at[pl.ds(c * npad + s * spt, spt)])

    return deg_kernel


def _make_agg_kernel(n, dh, nch, na):
    rpt = na // NS
    zr = min(rpt, 128)
    assert rpt % zr == 0 and nch % 2 == 0

    @functools.partial(
        pl.kernel,
        out_type=jax.ShapeDtypeStruct((NC, na, dh), jnp.float32),
        mesh=_mesh,
        compiler_params=pltpu.CompilerParams(use_tc_tiling_on_sc=False),
        scratch_types=[
            pltpu.VMEM((2, nch, K), jnp.int32),
            pltpu.VMEM((2, K, dh), jnp.float32),
            pltpu.VMEM((zr, dh), jnp.float32),
            pltpu.VMEM_SHARED((na, dh), jnp.float32),
            pltpu.SemaphoreType.DMA,
            pltpu.SemaphoreType.DMA,
        ],
    )
    def agg_kernel(hp_hbm, src_hbm, dst_hbm, out_hbm,
                   idx_v, rows_v, z_v, acc, sem0, sem1):
        c = lax.axis_index("c")
        s = lax.axis_index("s")
        pltpu.sync_copy(src_hbm.at[s], idx_v.at[0])
        pltpu.sync_copy(dst_hbm.at[s], idx_v.at[1])

        def zrow(i, carry):
            for jv in range(dh // 16):
                z_v[i, pl.ds(jv * 16, 16)] = jnp.zeros((16,), jnp.float32)
            return carry

        lax.fori_loop(0, zr, zrow, 0)
        for t in range(rpt // zr):
            pltpu.sync_copy(z_v, acc.at[pl.ds(s * rpt + t * zr, zr)])
        plsc.subcore_barrier()

        sems = (sem0, sem1)

        def run_core(hpc):
            pltpu.async_copy(hpc.at[idx_v.at[0, 0]], rows_v.at[0], sem0)

            def pair(i, carry):
                for b in range(2):
                    jj = 2 * i + b

                    @pl.when(jj + 1 < nch)
                    def _():
                        pltpu.async_copy(hpc.at[idx_v.at[0, jj + 1]],
                                         rows_v.at[1 - b], sems[1 - b])

                    pltpu.make_async_copy(hpc.at[idx_v.at[0, jj]],
                                          rows_v.at[b], sems[b]).wait()
                    pltpu.sync_copy(rows_v.at[b], acc.at[idx_v.at[1, jj]],
                                    add=True)
                return carry

            lax.fori_loop(0, nch // 2, pair, 0)

        for cc in range(NC):
            @pl.when(c == cc)
            def _():
                run_core(hp_hbm.at[cc])

        plsc.subcore_barrier()
        pltpu.sync_copy(acc.at[pl.ds(s * rpt, rpt)],
                        out_hbm.at[c, pl.ds(s * rpt, rpt)])

    return agg_kernel


def _mm_scale_body(x_ref, w_ref, dinv_ref, o_ref):
    x = x_ref[...]
    for h in range(2):
        o_ref[h] = jnp.dot(x, w_ref[h], preferred_element_type=jnp.float32) \
            * dinv_ref[...]


def _layer_body(sp_ref, hp_ref, dinv_ref, b_ref, wq_ref, o_ref):
    dinv = dinv_ref[...]
    z = [jnp.maximum((sp_ref[g] + hp_ref[g]) * dinv + b_ref[g], 0.0)
         for g in range(2)]
    for h in range(2):
        acc = jnp.dot(z[0], wq_ref[0, h], preferred_element_type=jnp.float32)
        acc += jnp.dot(z[1], wq_ref[1, h], preferred_element_type=jnp.float32)
        o_ref[h] = acc * dinv


def _final_body(sp_ref, hp_ref, dinv_ref, b_ref, o_ref):
    dinv = dinv_ref[...]
    for h in range(2):
        o_ref[:, pl.ds(64 * h, 64)] = (sp_ref[h] + hp_ref[h]) * dinv \
            + b_ref[h]


def _half_spec(r, dh):
    return pl.BlockSpec((2, r, dh), lambda i: (0, i, 0))


def _row_spec(r, d):
    return pl.BlockSpec((r, d), lambda i: (i, 0))


def _full_spec(shape):
    return pl.BlockSpec(shape, lambda i: tuple(0 for _ in shape))


def kernel(x, edge_index, W1, b1, W2, b2, W3, b3):
    n, d_in = x.shape
    hid = W1.shape[1]
    out_d = W3.shape[1]
    e = edge_index.shape[1]
    dh = hid // 2
    nchd = e // (NW * K)
    nch = e // (NS * K)
    assert nchd * NW * K == e and hid == out_d == 2 * dh

    src_d = edge_index[0].reshape(NS, nch, K)
    dst_d = edge_index[1].reshape(NS, nch, K)
    dst_w = edge_index[1].reshape(NW, nchd, K)

    npad = ((n + 16 * NS - 1) // (16 * NS)) * (16 * NS)
    degp = _make_deg_kernel(nchd, npad)(dst_w)
    deg = 1.0 + degp[:n] + degp[npad:npad + n]
    dinv = lax.rsqrt(deg).reshape(n, 1)

    agg = _make_agg_kernel(n, dh, nch, npad)

    r = 1000 if n % 1000 == 0 else n
    grid = (n // r,)

    w1h = W1.reshape(d_in, 2, dh).transpose(1, 0, 2)
    w2q = W2.reshape(2, dh, 2, dh).transpose(0, 2, 1, 3)
    w3q = W3.reshape(2, dh, 2, dh).transpose(0, 2, 1, 3)
    b1h = b1.reshape(2, 1, dh)
    b2h = b2.reshape(2, 1, dh)
    b3h = b3.reshape(2, 1, dh)

    mm_scale = pl.pallas_call(
        _mm_scale_body,
        grid=grid,
        in_specs=[_row_spec(r, d_in), _full_spec((2, d_in, dh)),
                  _row_spec(r, 1)],
        out_specs=_half_spec(r, dh),
        out_shape=jax.ShapeDtypeStruct((2, n, dh), jnp.float32),
    )

    layer = pl.pallas_call(
        _layer_body,
        grid=grid,
        in_specs=[_half_spec(r, dh), _half_spec(r, dh), _row_spec(r, 1),
                  _full_spec((2, 1, dh)), _full_spec((2, 2, dh, dh))],
        out_specs=_half_spec(r, dh),
        out_shape=jax.ShapeDtypeStruct((2, n, dh), jnp.float32),
    )

    final = pl.pallas_call(
        _final_body,
        grid=grid,
        in_specs=[_half_spec(r, dh), _half_spec(r, dh), _row_spec(r, 1),
                  _full_spec((2, 1, dh))],
        out_specs=_row_spec(r, out_d),
        out_shape=jax.ShapeDtypeStruct((n, out_d), jnp.float32),
    )

    h1p = mm_scale(x, w1h, dinv)
    s1 = agg(h1p, src_d, dst_d)
    h2p = layer(s1, h1p, dinv, b1h, w2q)
    s2 = agg(h2p, src_d, dst_d)
    h3p = layer(s2, h2p, dinv, b2h, w3q)
    s3 = agg(h3p, src_d, dst_d)
    return final(s3, h3p, dinv, b3h)


# speedup vs baseline: 28.1370x; 1.3679x over previous
import functools
import jax
import jax.numpy as jnp
from jax import lax
from jax.experimental import pallas as pl
from jax.experimental.pallas import tpu as pltpu
from jax.experimental.pallas import tpu_sc as plsc

NC = 2
NS = 16
NW = NC * NS
KD = 80
KA = 125

_mesh = plsc.VectorSubcoreMesh(core_axis_name="c", subcore_axis_name="s")


def _make_deg_kernel(nch, npad):
    spt = npad // NS
    K = KD

    @functools.partial(
        pl.kernel,
        out_type=jax.ShapeDtypeStruct((NC * npad,), jnp.float32),
        mesh=_mesh,
        scratch_types=[
            pltpu.VMEM((nch, K), jnp.int32),
            pltpu.VMEM((K,), jnp.float32),
            pltpu.VMEM((spt,), jnp.float32),
            pltpu.VMEM_SHARED((npad,), jnp.float32),
        ],
    )
    def deg_kernel(dst_hbm, out_hbm, idx_v, ones_v, z_v, acc):
        c = lax.axis_index("c")
        s = lax.axis_index("s")
        w = s * NC + c
        pltpu.sync_copy(dst_hbm.at[w], idx_v)

        for i in range(K // 16):
            ones_v[pl.ds(i * 16, 16)] = jnp.ones((16,), jnp.float32)

        def zrow(i, carry):
            z_v[pl.ds(i * 16, 16)] = jnp.zeros((16,), jnp.float32)
            return carry

        lax.fori_loop(0, spt // 16, zrow, 0)
        pltpu.sync_copy(z_v, acc.at[pl.ds(s * spt, spt)])
        plsc.subcore_barrier()

        def step(j, carry):
            pltpu.sync_copy(ones_v, acc.at[idx_v.at[j]], add=True)
            return carry

        lax.fori_loop(0, nch, step, 0)
        plsc.subcore_barrier()
        pltpu.sync_copy(acc.at[pl.ds(s * spt, spt)],
                        out_hbm.at[pl.ds(c * npad + s * spt, spt)])

    return deg_kernel


def _make_agg_kernel(n, dh, nch, na):
    rpt = na // NS
    zr = min(rpt, 128)
    nbuf = 5
    pd = 3
    K = KA
    assert rpt % zr == 0 and nch % nbuf == 0 and nch > nbuf

    @functools.partial(
        pl.kernel,
        out_type=jax.ShapeDtypeStruct((NC, na, dh), jnp.float32),
        mesh=_mesh,
        compiler_params=pltpu.CompilerParams(use_tc_tiling_on_sc=False),
        scratch_types=[
            pltpu.VMEM((2, nch, K), jnp.int32),
            pltpu.VMEM((nbuf, K, dh), jnp.float32),
            pltpu.VMEM((zr, dh), jnp.float32),
            pltpu.VMEM_SHARED((na, dh), jnp.float32),
            [pltpu.SemaphoreType.DMA] * nbuf,
            [pltpu.SemaphoreType.DMA] * nbuf,
        ],
    )
    def agg_kernel(hp_hbm, src_hbm, dst_hbm, out_hbm,
                   idx_v, rows_v, z_v, acc, gsem, ssem):
        c = lax.axis_index("c")
        s = lax.axis_index("s")
        pltpu.sync_copy(src_hbm.at[s], idx_v.at[0])
        pltpu.sync_copy(dst_hbm.at[s], idx_v.at[1])

        def zrow(i, carry):
            for jv in range(dh // 16):
                z_v[i, pl.ds(jv * 16, 16)] = jnp.zeros((16,), jnp.float32)
            return carry

        lax.fori_loop(0, zr, zrow, 0)
        for t in range(rpt // zr):
            pltpu.sync_copy(z_v, acc.at[pl.ds(s * rpt + t * zr, zr)])
        plsc.subcore_barrier()

        def run_core(hpc):
            for b in range(pd):
                pltpu.async_copy(hpc.at[idx_v.at[0, b]], rows_v.at[b],
                                 gsem[b])

            def ring(i, carry):
                for b in range(nbuf):
                    jj = nbuf * i + b
                    pltpu.make_async_copy(hpc.at[idx_v.at[0, jj]],
                                          rows_v.at[b], gsem[b]).wait()
                    pltpu.async_copy(rows_v.at[b], acc.at[idx_v.at[1, jj]],
                                     ssem[b], add=True)
                    bn = (b + pd) % nbuf

                    @pl.when(jj >= nbuf - pd)
                    def _():
                        pltpu.make_async_copy(
                            rows_v.at[bn],
                            acc.at[idx_v.at[1, jj - (nbuf - pd)]],
                            ssem[bn]).wait()

                    @pl.when(jj + pd < nch)
                    def _():
                        pltpu.async_copy(hpc.at[idx_v.at[0, jj + pd]],
                                         rows_v.at[bn], gsem[bn])
                return carry

            lax.fori_loop(0, nch // nbuf, ring, 0)
            for t in range(nbuf - pd):
                jj = nch - (nbuf - pd) + t
                pltpu.make_async_copy(rows_v.at[jj % nbuf],
                                      acc.at[idx_v.at[1, jj]],
                                      ssem[jj % nbuf]).wait()

        for cc in range(NC):
            @pl.when(c == cc)
            def _():
                run_core(hp_hbm.at[cc])

        plsc.subcore_barrier()
        pltpu.sync_copy(acc.at[pl.ds(s * rpt, rpt)],
                        out_hbm.at[c, pl.ds(s * rpt, rpt)])

    return agg_kernel


def _mm_scale_body(x_ref, w_ref, dinv_ref, o_ref):
    x = x_ref[...]
    for h in range(2):
        o_ref[h] = jnp.dot(x, w_ref[h], preferred_element_type=jnp.float32) \
            * dinv_ref[...]


def _layer_body(sp_ref, hp_ref, dinv_ref, b_ref, wq_ref, o_ref):
    dinv = dinv_ref[...]
    z = [jnp.maximum((sp_ref[g] + hp_ref[g]) * dinv + b_ref[g], 0.0)
         for g in range(2)]
    for h in range(2):
        acc = jnp.dot(z[0], wq_ref[0, h], preferred_element_type=jnp.float32)
        acc += jnp.dot(z[1], wq_ref[1, h], preferred_element_type=jnp.float32)
        o_ref[h] = acc * dinv


def _final_body(sp_ref, hp_ref, dinv_ref, b_ref, o_ref):
    dinv = dinv_ref[...]
    for h in range(2):
        o_ref[:, pl.ds(64 * h, 64)] = (sp_ref[h] + hp_ref[h]) * dinv \
            + b_ref[h]


def _half_spec(r, dh):
    return pl.BlockSpec((2, r, dh), lambda i: (0, i, 0))


def _row_spec(r, d):
    return pl.BlockSpec((r, d), lambda i: (i, 0))


def _full_spec(shape):
    return pl.BlockSpec(shape, lambda i: tuple(0 for _ in shape))


def kernel(x, edge_index, W1, b1, W2, b2, W3, b3):
    n, d_in = x.shape
    hid = W1.shape[1]
    out_d = W3.shape[1]
    e = edge_index.shape[1]
    dh = hid // 2
    nchd = e // (NW * KD)
    nch = e // (NS * KA)
    assert nchd * NW * KD == e and nch * NS * KA == e
    assert hid == out_d == 2 * dh

    src_d = edge_index[0].reshape(NS, nch, KA)
    dst_d = edge_index[1].reshape(NS, nch, KA)
    dst_w = edge_index[1].reshape(NW, nchd, KD)

    npad = ((n + 16 * NS - 1) // (16 * NS)) * (16 * NS)
    degp = _make_deg_kernel(nchd, npad)(dst_w)
    deg = 1.0 + degp[:n] + degp[npad:npad + n]
    dinv = lax.rsqrt(deg).reshape(n, 1)

    agg = _make_agg_kernel(n, dh, nch, npad)

    r = 1000 if n % 1000 == 0 else n
    grid = (n // r,)

    w1h = W1.reshape(d_in, 2, dh).transpose(1, 0, 2)
    w2q = W2.reshape(2, dh, 2, dh).transpose(0, 2, 1, 3)
    w3q = W3.reshape(2, dh, 2, dh).transpose(0, 2, 1, 3)
    b1h = b1.reshape(2, 1, dh)
    b2h = b2.reshape(2, 1, dh)
    b3h = b3.reshape(2, 1, dh)

    mm_scale = pl.pallas_call(
        _mm_scale_body,
        grid=grid,
        in_specs=[_row_spec(r, d_in), _full_spec((2, d_in, dh)),
                  _row_spec(r, 1)],
        out_specs=_half_spec(r, dh),
        out_shape=jax.ShapeDtypeStruct((2, n, dh), jnp.float32),
    )

    layer = pl.pallas_call(
        _layer_body,
        grid=grid,
        in_specs=[_half_spec(r, dh), _half_spec(r, dh), _row_spec(r, 1),
                  _full_spec((2, 1, dh)), _full_spec((2, 2, dh, dh))],
        out_specs=_half_spec(r, dh),
        out_shape=jax.ShapeDtypeStruct((2, n, dh), jnp.float32),
    )

    final = pl.pallas_call(
        _final_body,
        grid=grid,
        in_specs=[_half_spec(r, dh), _half_spec(r, dh), _row_spec(r, 1),
                  _full_spec((2, 1, dh))],
        out_specs=_row_spec(r, out_d),
        out_shape=jax.ShapeDtypeStruct((n, out_d), jnp.float32),
    )

    h1p = mm_scale(x, w1h, dinv)
    s1 = agg(h1p, src_d, dst_d)
    h2p = layer(s1, h1p, dinv, b1h, w2q)
    s2 = agg(h2p, src_d, dst_d)
    h3p = layer(s2, h2p, dinv, b2h, w3q)
    s3 = agg(h3p, src_d, dst_d)
    return final(s3, h3p, dinv, b3h)


# speedup vs baseline: 29.1206x; 1.0350x over previous
import functools
import jax
import jax.numpy as jnp
from jax import lax
from jax.experimental import pallas as pl
from jax.experimental.pallas import tpu as pltpu
from jax.experimental.pallas import tpu_sc as plsc

NC = 2
NS = 16
NW = NC * NS
KD = 80
KA = 125

_mesh = plsc.VectorSubcoreMesh(core_axis_name="c", subcore_axis_name="s")


def _make_deg_kernel(nch, npad):
    spt = npad // NS
    K = KD

    @functools.partial(
        pl.kernel,
        out_type=jax.ShapeDtypeStruct((NC * npad,), jnp.float32),
        mesh=_mesh,
        scratch_types=[
            pltpu.VMEM((nch, K), jnp.int32),
            pltpu.VMEM((K,), jnp.float32),
            pltpu.VMEM((spt,), jnp.float32),
            pltpu.VMEM_SHARED((npad,), jnp.float32),
        ],
    )
    def deg_kernel(dst_hbm, out_hbm, idx_v, ones_v, z_v, acc):
        c = lax.axis_index("c")
        s = lax.axis_index("s")
        w = s * NC + c
        pltpu.sync_copy(dst_hbm.at[w], idx_v)

        for i in range(K // 16):
            ones_v[pl.ds(i * 16, 16)] = jnp.ones((16,), jnp.float32)

        def zrow(i, carry):
            z_v[pl.ds(i * 16, 16)] = jnp.zeros((16,), jnp.float32)
            return carry

        lax.fori_loop(0, spt // 16, zrow, 0)
        pltpu.sync_copy(z_v, acc.at[pl.ds(s * spt, spt)])
        plsc.subcore_barrier()

        def step(j, carry):
            pltpu.sync_copy(ones_v, acc.at[idx_v.at[j]], add=True)
            return carry

        lax.fori_loop(0, nch, step, 0)
        plsc.subcore_barrier()
        pltpu.sync_copy(acc.at[pl.ds(s * spt, spt)],
                        out_hbm.at[pl.ds(c * npad + s * spt, spt)])

    return deg_kernel


def _make_agg_kernel(n, dh, nch, na):
    rpt = na // NS
    zr = 80
    nbuf = 6
    pd = 4
    K = KA
    nch0 = (nch // nbuf) * nbuf
    assert rpt % zr == 0 and zr <= K and nch0 > nbuf

    @functools.partial(
        pl.kernel,
        out_type=jax.ShapeDtypeStruct((NC, na, dh), jnp.float32),
        mesh=_mesh,
        compiler_params=pltpu.CompilerParams(use_tc_tiling_on_sc=False),
        scratch_types=[
            pltpu.VMEM((2, nch, K), jnp.int32),
            pltpu.VMEM((nbuf, K, dh), jnp.float32),
            pltpu.VMEM_SHARED((na, dh), jnp.float32),
            [pltpu.SemaphoreType.DMA] * nbuf,
            [pltpu.SemaphoreType.DMA] * nbuf,
        ],
    )
    def agg_kernel(hp_hbm, src_hbm, dst_hbm, out_hbm,
                   idx_v, rows_v, acc, gsem, ssem):
        c = lax.axis_index("c")
        s = lax.axis_index("s")
        pltpu.sync_copy(src_hbm.at[s], idx_v.at[0])
        pltpu.sync_copy(dst_hbm.at[s], idx_v.at[1])

        def zrow(i, carry):
            for jv in range(dh // 16):
                rows_v[0, i, pl.ds(jv * 16, 16)] = jnp.zeros((16,),
                                                             jnp.float32)
            return carry

        lax.fori_loop(0, zr, zrow, 0)
        for t in range(rpt // zr):
            pltpu.sync_copy(rows_v.at[0, pl.ds(0, zr)],
                            acc.at[pl.ds(s * rpt + t * zr, zr)])
        plsc.subcore_barrier()

        def chunk_step(jj, b, static):
            pltpu.make_async_copy(hpc_ref[0].at[idx_v.at[0, jj]],
                                  rows_v.at[b], gsem[b]).wait()
            pltpu.async_copy(rows_v.at[b], acc.at[idx_v.at[1, jj]],
                             ssem[b], add=True)
            bn = (b + pd) % nbuf

            def wait_prev():
                pltpu.make_async_copy(
                    rows_v.at[bn],
                    acc.at[idx_v.at[1, jj - (nbuf - pd)]],
                    ssem[bn]).wait()

            def fetch_next():
                pltpu.async_copy(hpc_ref[0].at[idx_v.at[0, jj + pd]],
                                 rows_v.at[bn], gsem[bn])

            if static:
                if jj >= nbuf - pd:
                    wait_prev()
                if jj + pd < nch:
                    fetch_next()
            else:
                pl.when(jj >= nbuf - pd)(wait_prev)
                pl.when(jj + pd < nch)(fetch_next)

        hpc_ref = [None]

        def run_core(hpc):
            hpc_ref[0] = hpc
            for b in range(pd):
                pltpu.async_copy(hpc.at[idx_v.at[0, b]], rows_v.at[b],
                                 gsem[b])

            def ring(i, carry):
                for b in range(nbuf):
                    chunk_step(nbuf * i + b, b, static=False)
                return carry

            lax.fori_loop(0, nch0 // nbuf, ring, 0)
            for jj in range(nch0, nch):
                chunk_step(jj, jj % nbuf, static=True)
            for t in range(nbuf - pd):
                jj = nch - (nbuf - pd) + t
                pltpu.make_async_copy(rows_v.at[jj % nbuf],
                                      acc.at[idx_v.at[1, jj]],
                                      ssem[jj % nbuf]).wait()

        for cc in range(NC):
            @pl.when(c == cc)
            def _():
                run_core(hp_hbm.at[cc])

        plsc.subcore_barrier()
        pltpu.sync_copy(acc.at[pl.ds(s * rpt, rpt)],
                        out_hbm.at[c, pl.ds(s * rpt, rpt)])

    return agg_kernel


def _mm_scale_body(x_ref, w_ref, dinv_ref, o_ref):
    x = x_ref[...]
    for h in range(2):
        o_ref[h] = jnp.dot(x, w_ref[h], preferred_element_type=jnp.float32) \
            * dinv_ref[...]


def _layer_body(sp_ref, hp_ref, dinv_ref, b_ref, wq_ref, o_ref):
    dinv = dinv_ref[...]
    z = [jnp.maximum((sp_ref[g] + hp_ref[g]) * dinv + b_ref[g], 0.0)
         for g in range(2)]
    for h in range(2):
        acc = jnp.dot(z[0], wq_ref[0, h], preferred_element_type=jnp.float32)
        acc += jnp.dot(z[1], wq_ref[1, h], preferred_element_type=jnp.float32)
        o_ref[h] = acc * dinv


def _final_body(sp_ref, hp_ref, dinv_ref, b_ref, o_ref):
    dinv = dinv_ref[...]
    for h in range(2):
        o_ref[:, pl.ds(64 * h, 64)] = (sp_ref[h] + hp_ref[h]) * dinv \
            + b_ref[h]


def _half_spec(r, dh):
    return pl.BlockSpec((2, r, dh), lambda i: (0, i, 0))


def _row_spec(r, d):
    return pl.BlockSpec((r, d), lambda i: (i, 0))


def _full_spec(shape):
    return pl.BlockSpec(shape, lambda i: tuple(0 for _ in shape))


def kernel(x, edge_index, W1, b1, W2, b2, W3, b3):
    n, d_in = x.shape
    hid = W1.shape[1]
    out_d = W3.shape[1]
    e = edge_index.shape[1]
    dh = hid // 2
    nchd = e // (NW * KD)
    nch = e // (NS * KA)
    assert nchd * NW * KD == e and nch * NS * KA == e
    assert hid == out_d == 2 * dh

    src_d = edge_index[0].reshape(NS, nch, KA)
    dst_d = edge_index[1].reshape(NS, nch, KA)
    dst_w = edge_index[1].reshape(NW, nchd, KD)

    npad = ((n + 16 * NS - 1) // (16 * NS)) * (16 * NS)
    degp = _make_deg_kernel(nchd, npad)(dst_w)
    deg = 1.0 + degp[:n] + degp[npad:npad + n]
    dinv = lax.rsqrt(deg).reshape(n, 1)

    agg = _make_agg_kernel(n, dh, nch, npad)

    r = 1000 if n % 1000 == 0 else n
    grid = (n // r,)

    w1h = W1.reshape(d_in, 2, dh).transpose(1, 0, 2)
    w2q = W2.reshape(2, dh, 2, dh).transpose(0, 2, 1, 3)
    w3q = W3.reshape(2, dh, 2, dh).transpose(0, 2, 1, 3)
    b1h = b1.reshape(2, 1, dh)
    b2h = b2.reshape(2, 1, dh)
    b3h = b3.reshape(2, 1, dh)

    mm_scale = pl.pallas_call(
        _mm_scale_body,
        grid=grid,
        in_specs=[_row_spec(r, d_in), _full_spec((2, d_in, dh)),
                  _row_spec(r, 1)],
        out_specs=_half_spec(r, dh),
        out_shape=jax.ShapeDtypeStruct((2, n, dh), jnp.float32),
    )

    layer = pl.pallas_call(
        _layer_body,
        grid=grid,
        in_specs=[_half_spec(r, dh), _half_spec(r, dh), _row_spec(r, 1),
                  _full_spec((2, 1, dh)), _full_spec((2, 2, dh, dh))],
        out_specs=_half_spec(r, dh),
        out_shape=jax.ShapeDtypeStruct((2, n, dh), jnp.float32),
    )

    final = pl.pallas_call(
        _final_body,
        grid=grid,
        in_specs=[_half_spec(r, dh), _half_spec(r, dh), _row_spec(r, 1),
                  _full_spec((2, 1, dh))],
        out_specs=_row_spec(r, out_d),
        out_shape=jax.ShapeDtypeStruct((n, out_d), jnp.float32),
    )

    h1p = mm_scale(x, w1h, dinv)
    s1 = agg(h1p, src_d, dst_d)
    h2p = layer(s1, h1p, dinv, b1h, w2q)
    s2 = agg(h2p, src_d, dst_d)
    h3p = layer(s2, h2p, dinv, b2h, w3q)
    s3 = agg(h3p, src_d, dst_d)
    return final(s3, h3p, dinv, b3h)


# speedup vs baseline: 29.1447x; 1.0008x over previous
import functools
import jax
import jax.numpy as jnp
from jax import lax
from jax.experimental import pallas as pl
from jax.experimental.pallas import tpu as pltpu
from jax.experimental.pallas import tpu_sc as plsc

NC = 2
NS = 16
NW = NC * NS
KD = 80
KA = 125

_mesh = plsc.VectorSubcoreMesh(core_axis_name="c", subcore_axis_name="s")


def _make_deg_kernel(nch, npad):
    spt = npad // NS
    K = KD

    @functools.partial(
        pl.kernel,
        out_type=jax.ShapeDtypeStruct((NC * npad,), jnp.float32),
        mesh=_mesh,
        scratch_types=[
            pltpu.VMEM((nch, K), jnp.int32),
            pltpu.VMEM((K,), jnp.float32),
            pltpu.VMEM((spt,), jnp.float32),
            pltpu.VMEM_SHARED((npad,), jnp.float32),
        ],
    )
    def deg_kernel(dst_hbm, out_hbm, idx_v, ones_v, z_v, acc):
        c = lax.axis_index("c")
        s = lax.axis_index("s")
        w = s * NC + c
        pltpu.sync_copy(dst_hbm.at[w], idx_v)

        for i in range(K // 16):
            ones_v[pl.ds(i * 16, 16)] = jnp.ones((16,), jnp.float32)

        def zrow(i, carry):
            z_v[pl.ds(i * 16, 16)] = jnp.zeros((16,), jnp.float32)
            return carry

        lax.fori_loop(0, spt // 16, zrow, 0)
        pltpu.sync_copy(z_v, acc.at[pl.ds(s * spt, spt)])
        plsc.subcore_barrier()

        def step(j, carry):
            pltpu.sync_copy(ones_v, acc.at[idx_v.at[j]], add=True)
            return carry

        lax.fori_loop(0, nch, step, 0)
        plsc.subcore_barrier()
        pltpu.sync_copy(acc.at[pl.ds(s * spt, spt)],
                        out_hbm.at[pl.ds(c * npad + s * spt, spt)])

    return deg_kernel


def _make_agg_kernel(n, dh, nch, na):
    rpt = na // NS
    zr = 80
    nbuf = 6
    pd = 5
    K = KA
    nch0 = (nch // nbuf) * nbuf
    assert rpt % zr == 0 and zr <= K and nch0 > nbuf

    @functools.partial(
        pl.kernel,
        out_type=jax.ShapeDtypeStruct((NC, na, dh), jnp.float32),
        mesh=_mesh,
        compiler_params=pltpu.CompilerParams(use_tc_tiling_on_sc=False),
        scratch_types=[
            pltpu.VMEM((2, nch, K), jnp.int32),
            pltpu.VMEM((nbuf, K, dh), jnp.float32),
            pltpu.VMEM_SHARED((na, dh), jnp.float32),
            [pltpu.SemaphoreType.DMA] * nbuf,
            [pltpu.SemaphoreType.DMA] * nbuf,
        ],
    )
    def agg_kernel(hp_hbm, src_hbm, dst_hbm, out_hbm,
                   idx_v, rows_v, acc, gsem, ssem):
        c = lax.axis_index("c")
        s = lax.axis_index("s")
        pltpu.sync_copy(src_hbm.at[s], idx_v.at[0])
        pltpu.sync_copy(dst_hbm.at[s], idx_v.at[1])

        def zrow(i, carry):
            for jv in range(dh // 16):
                rows_v[0, i, pl.ds(jv * 16, 16)] = jnp.zeros((16,),
                                                             jnp.float32)
            return carry

        lax.fori_loop(0, zr, zrow, 0)
        for t in range(rpt // zr):
            pltpu.sync_copy(rows_v.at[0, pl.ds(0, zr)],
                            acc.at[pl.ds(s * rpt + t * zr, zr)])
        plsc.subcore_barrier()

        def chunk_step(jj, b, static):
            pltpu.make_async_copy(hpc_ref[0].at[idx_v.at[0, jj]],
                                  rows_v.at[b], gsem[b]).wait()
            pltpu.async_copy(rows_v.at[b], acc.at[idx_v.at[1, jj]],
                             ssem[b], add=True)
            bn = (b + pd) % nbuf

            def wait_prev():
                pltpu.make_async_copy(
                    rows_v.at[bn],
                    acc.at[idx_v.at[1, jj - (nbuf - pd)]],
                    ssem[bn]).wait()

            def fetch_next():
                pltpu.async_copy(hpc_ref[0].at[idx_v.at[0, jj + pd]],
                                 rows_v.at[bn], gsem[bn])

            if static:
                if jj >= nbuf - pd:
                    wait_prev()
                if jj + pd < nch:
                    fetch_next()
            else:
                pl.when(jj >= nbuf - pd)(wait_prev)
                pl.when(jj + pd < nch)(fetch_next)

        hpc_ref = [None]

        def run_core(hpc):
            hpc_ref[0] = hpc
            for b in range(pd):
                pltpu.async_copy(hpc.at[idx_v.at[0, b]], rows_v.at[b],
                                 gsem[b])

            def ring(i, carry):
                for b in range(nbuf):
                    chunk_step(nbuf * i + b, b, static=False)
                return carry

            lax.fori_loop(0, nch0 // nbuf, ring, 0)
            for jj in range(nch0, nch):
                chunk_step(jj, jj % nbuf, static=True)
            for t in range(nbuf - pd):
                jj = nch - (nbuf - pd) + t
                pltpu.make_async_copy(rows_v.at[jj % nbuf],
                                      acc.at[idx_v.at[1, jj]],
                                      ssem[jj % nbuf]).wait()

        for cc in range(NC):
            @pl.when(c == cc)
            def _():
                run_core(hp_hbm.at[cc])

        plsc.subcore_barrier()
        pltpu.sync_copy(acc.at[pl.ds(s * rpt, rpt)],
                        out_hbm.at[c, pl.ds(s * rpt, rpt)])

    return agg_kernel


def _mm_scale_body(x_ref, w_ref, dinv_ref, o_ref):
    x = x_ref[...]
    for h in range(2):
        o_ref[h] = jnp.dot(x, w_ref[h], preferred_element_type=jnp.float32) \
            * dinv_ref[...]


def _layer_body(sp_ref, hp_ref, dinv_ref, b_ref, wq_ref, o_ref):
    dinv = dinv_ref[...]
    z = [jnp.maximum((sp_ref[g] + hp_ref[g]) * dinv + b_ref[g], 0.0)
         for g in range(2)]
    for h in range(2):
        acc = jnp.dot(z[0], wq_ref[0, h], preferred_element_type=jnp.float32)
        acc += jnp.dot(z[1], wq_ref[1, h], preferred_element_type=jnp.float32)
        o_ref[h] = acc * dinv


def _final_body(sp_ref, hp_ref, dinv_ref, b_ref, o_ref):
    dinv = dinv_ref[...]
    for h in range(2):
        o_ref[:, pl.ds(64 * h, 64)] = (sp_ref[h] + hp_ref[h]) * dinv \
            + b_ref[h]


def _half_spec(r, dh):
    return pl.BlockSpec((2, r, dh), lambda i: (0, i, 0))


def _row_spec(r, d):
    return pl.BlockSpec((r, d), lambda i: (i, 0))


def _full_spec(shape):
    return pl.BlockSpec(shape, lambda i: tuple(0 for _ in shape))


def kernel(x, edge_index, W1, b1, W2, b2, W3, b3):
    n, d_in = x.shape
    hid = W1.shape[1]
    out_d = W3.shape[1]
    e = edge_index.shape[1]
    dh = hid // 2
    nchd = e // (NW * KD)
    nch = e // (NS * KA)
    assert nchd * NW * KD == e and nch * NS * KA == e
    assert hid == out_d == 2 * dh

    src_d = edge_index[0].reshape(NS, nch, KA)
    dst_d = edge_index[1].reshape(NS, nch, KA)
    dst_w = edge_index[1].reshape(NW, nchd, KD)

    npad = ((n + 16 * NS - 1) // (16 * NS)) * (16 * NS)
    degp = _make_deg_kernel(nchd, npad)(dst_w)
    deg = 1.0 + degp[:n] + degp[npad:npad + n]
    dinv = lax.rsqrt(deg).reshape(n, 1)

    agg = _make_agg_kernel(n, dh, nch, npad)

    r = 1000 if n % 1000 == 0 else n
    grid = (n // r,)

    w1h = W1.reshape(d_in, 2, dh).transpose(1, 0, 2)
    w2q = W2.reshape(2, dh, 2, dh).transpose(0, 2, 1, 3)
    w3q = W3.reshape(2, dh, 2, dh).transpose(0, 2, 1, 3)
    b1h = b1.reshape(2, 1, dh)
    b2h = b2.reshape(2, 1, dh)
    b3h = b3.reshape(2, 1, dh)

    mm_scale = pl.pallas_call(
        _mm_scale_body,
        grid=grid,
        in_specs=[_row_spec(r, d_in), _full_spec((2, d_in, dh)),
                  _row_spec(r, 1)],
        out_specs=_half_spec(r, dh),
        out_shape=jax.ShapeDtypeStruct((2, n, dh), jnp.float32),
    )

    layer = pl.pallas_call(
        _layer_body,
        grid=grid,
        in_specs=[_half_spec(r, dh), _half_spec(r, dh), _row_spec(r, 1),
                  _full_spec((2, 1, dh)), _full_spec((2, 2, dh, dh))],
        out_specs=_half_spec(r, dh),
        out_shape=jax.ShapeDtypeStruct((2, n, dh), jnp.float32),
    )

    final = pl.pallas_call(
        _final_body,
        grid=grid,
        in_specs=[_half_spec(r, dh), _half_spec(r, dh), _row_spec(r, 1),
                  _full_spec((2, 1, dh))],
        out_specs=_row_spec(r, out_d),
        out_shape=jax.ShapeDtypeStruct((n, out_d), jnp.float32),
    )

    h1p = mm_scale(x, w1h, dinv)
    s1 = agg(h1p, src_d, dst_d)
    h2p = layer(s1, h1p, dinv, b1h, w2q)
    s2 = agg(h2p, src_d, dst_d)
    h3p = layer(s2, h2p, dinv, b2h, w3q)
    s3 = agg(h3p, src_d, dst_d)
    return final(s3, h3p, dinv, b3h)
